# baseline (speedup 1.0000x reference)
import functools

import jax
import jax.numpy as jnp
from jax import lax
from jax.experimental import pallas as pl
from jax.experimental.pallas import tpu as pltpu
from jax.experimental.pallas import tpu_sc as plsc

_NW = 32
_NP = 4


def _scale_kernel(sp_ref, w_ref, scale_ref):
    xf = jax.lax.dot_general(
        sp_ref[...], w_ref[...],
        dimension_numbers=(((1,), (1,)), ((), ())),
        preferred_element_type=jnp.float32,
    )
    scale_ref[...] = 1.0 + jax.nn.sigmoid(xf)


def _make_sc_mul(n, hw, c):
    mesh = plsc.VectorSubcoreMesh(core_axis_name="c", subcore_axis_name="s")

    @functools.partial(
        pl.kernel,
        mesh=mesh,
        out_type=jax.ShapeDtypeStruct((n, hw, c), jnp.float32),
        scratch_types=[
            pltpu.VMEM((_NP, c), jnp.float32),
            pltpu.VMEM((2, _NP, c), jnp.float32),
            pltpu.VMEM((2, _NP, c), jnp.float32),
            pltpu.SemaphoreType.DMA((2,)),
            pltpu.SemaphoreType.DMA((2,)),
        ],
    )
    def sc_mul(scale_hbm, x_hbm, out_hbm, scale_v, xb, ob, in_sems, out_sems):
        cc = lax.axis_index("c")
        ss = lax.axis_index("s")
        wid = ss * 2 + cc
        p0 = wid * _NP

        pltpu.make_async_copy(
            scale_hbm.at[pl.ds(p0, _NP)], scale_v, in_sems.at[0]
        ).start()
        pltpu.make_async_copy(
            scale_hbm.at[pl.ds(p0, _NP)], scale_v, in_sems.at[0]
        ).wait()

        def in_copy(b, slot):
            return pltpu.make_async_copy(
                x_hbm.at[b, pl.ds(p0, _NP)], xb.at[slot], in_sems.at[slot]
            )

        def out_copy(b, slot):
            return pltpu.make_async_copy(
                ob.at[slot], out_hbm.at[b, pl.ds(p0, _NP)], out_sems.at[slot]
            )

        in_copy(0, 0).start()
        in_copy(1, 1).start()

        def body(g, carry):
            for sl in range(2):
                b = 2 * g + sl
                in_copy(b, sl).wait()

                @pl.when(b >= 2)
                def _():
                    out_copy(b - 2, sl).wait()

                def inner(i, carry2):
                    for r in range(_NP):
                        ob[sl, r, pl.ds(i * 16, 16)] = (
                            xb[sl, r, pl.ds(i * 16, 16)]
                            * scale_v[r, pl.ds(i * 16, 16)]
                        )
                    return carry2

                lax.fori_loop(0, c // 16, inner, 0, unroll=4)
                out_copy(b, sl).start()

                @pl.when(b + 2 < n)
                def _():
                    in_copy(b + 2, sl).start()
            return carry

        lax.fori_loop(0, n // 2, body, 0)
        out_copy(n - 2, 0).wait()
        out_copy(n - 1, 1).wait()

    return sc_mul


def kernel(inputs, labels, cpct_r_w, conv_w, similar_prototype):
    n, c, h, w = inputs.shape
    hw = h * w
    x = inputs.transpose(0, 2, 3, 1).reshape(n, hw, c)
    sp = similar_prototype.transpose(1, 2, 0).reshape(hw, c)

    scale = pl.pallas_call(
        _scale_kernel,
        out_shape=jax.ShapeDtypeStruct((hw, c), jnp.float32),
    )(sp, conv_w)

    out = _make_sc_mul(n, hw, c)(scale, x)
    return out.reshape(n, h, w, c).transpose(0, 3, 1, 2)


# speedup vs baseline: 3.3835x; 3.3835x over previous
import jax
import jax.numpy as jnp
from jax.experimental import pallas as pl
from jax.experimental.pallas import tpu as pltpu

_BN = 8


def _aol_kernel(sp_ref, w_ref, x_ref, out_ref, scale_ref):
    @pl.when(pl.program_id(0) == 0)
    def _compute_scale():
        xf = jax.lax.dot_general(
            sp_ref[...], w_ref[...],
            dimension_numbers=(((1,), (1,)), ((), ())),
            preferred_element_type=jnp.float32,
        )
        scale_ref[...] = 1.0 + jax.nn.sigmoid(xf)

    out_ref[...] = x_ref[...] * scale_ref[...][None, :, :]


def kernel(inputs, labels, cpct_r_w, conv_w, similar_prototype):
    n, c, h, w = inputs.shape
    hw = h * w
    x = inputs.transpose(0, 2, 3, 1).reshape(n, hw, c)
    sp = similar_prototype.transpose(1, 2, 0).reshape(hw, c)

    out = pl.pallas_call(
        _aol_kernel,
        grid=(n // _BN,),
        in_specs=[
            pl.BlockSpec((hw, c), lambda i: (0, 0)),
            pl.BlockSpec((c, c), lambda i: (0, 0)),
            pl.BlockSpec((_BN, hw, c), lambda i: (i, 0, 0)),
        ],
        out_specs=pl.BlockSpec((_BN, hw, c), lambda i: (i, 0, 0)),
        out_shape=jax.ShapeDtypeStruct((n, hw, c), inputs.dtype),
        scratch_shapes=[pltpu.VMEM((hw, c), jnp.float32)],
    )(sp, conv_w, x)
    return out.reshape(n, h, w, c).transpose(0, 3, 1, 2)
